# Initial kernel scaffold; baseline (speedup 1.0000x reference)
import jax
import jax.numpy as jnp
from jax.experimental import pallas as pl


def kernel(bytes_flat, table, W, b):
    raise NotImplementedError("write your pallas kernel here")


# speedup vs baseline: 3.6277x; 3.6277x over previous
import functools

import jax
import jax.numpy as jnp
from jax import lax
from jax.experimental import pallas as pl
from jax.experimental.pallas import tpu as pltpu
from jax.experimental.pallas import tpu_sc as plsc

_PATCH = 8
_IDX_CHUNK = 128


@functools.lru_cache(maxsize=None)
def _make_sc_gather(num_idx: int, dim: int):
    info = plsc.get_sparse_core_info()
    nc, ns = info.num_cores, info.num_subcores
    nw = nc * ns
    rows_per_w = num_idx // nw
    chunks = rows_per_w // _IDX_CHUNK
    mesh = plsc.VectorSubcoreMesh(core_axis_name="c", subcore_axis_name="s")

    @functools.partial(
        pl.kernel,
        mesh=mesh,
        out_type=jax.ShapeDtypeStruct((num_idx, dim), jnp.float32),
        scratch_types=[
            pltpu.VMEM((chunks, _IDX_CHUNK), jnp.int32),
            pltpu.VMEM((rows_per_w, dim), jnp.float32),
            pltpu.SemaphoreType.DMA,
        ],
        compiler_params=pltpu.CompilerParams(use_tc_tiling_on_sc=False),
    )
    def gather(idx_hbm, table_hbm, out_hbm, idx_v, rows_v, sem):
        wid = lax.axis_index("s") * nc + lax.axis_index("c")
        pltpu.sync_copy(idx_hbm.at[pl.ds(wid * chunks, chunks)], idx_v)
        copies = []
        for ci in range(chunks):
            copies.append(
                pltpu.async_copy(
                    table_hbm.at[idx_v.at[ci]],
                    rows_v.at[pl.ds(ci * _IDX_CHUNK, _IDX_CHUNK)],
                    sem,
                )
            )
        for cp in copies:
            cp.wait()
        pltpu.sync_copy(rows_v, out_hbm.at[pl.ds(wid * rows_per_w, rows_per_w)])

    return gather


def _mm_body(x_ref, w_ref, b_ref, o_ref):
    o_ref[...] = (
        jnp.dot(x_ref[...], w_ref[...], preferred_element_type=jnp.float32)
        + b_ref[...]
    )


def _tc_matmul(x, w, b2d, bm):
    m, k = x.shape
    n = w.shape[1]
    return pl.pallas_call(
        _mm_body,
        grid=(m // bm,),
        in_specs=[
            pl.BlockSpec((bm, k), lambda i: (i, 0)),
            pl.BlockSpec((k, n), lambda i: (0, 0)),
            pl.BlockSpec((1, n), lambda i: (0, 0)),
        ],
        out_specs=pl.BlockSpec((bm, n), lambda i: (i, 0)),
        out_shape=jax.ShapeDtypeStruct((m, n), jnp.float32),
        compiler_params=pltpu.CompilerParams(
            dimension_semantics=("arbitrary",),
        ),
    )(x, w, b2d)


def kernel(bytes_flat, table, W, b):
    B, L = bytes_flat.shape
    P = _PATCH
    T = L // P
    byte_dim = table.shape[1]
    n_idx = B * T * P

    idx2d = bytes_flat[:, : T * P].reshape(n_idx // _IDX_CHUNK, _IDX_CHUNK)
    gather = _make_sc_gather(n_idx, byte_dim)
    embs = gather(idx2d, table)

    x = embs.reshape(B * T, P * byte_dim)
    out = _tc_matmul(x, W, b.reshape(1, -1), 512)
    return out.reshape(B, T, -1), T
